# baseline (speedup 1.0000x reference)
import functools

import jax
import jax.numpy as jnp
from jax import lax
from jax.experimental import pallas as pl
from jax.experimental.pallas import tpu as pltpu
from jax.experimental.pallas import tpu_sc as plsc

FEAT = 128
GATE3 = 3 * FEAT
DEG_P = 4
DEG_C = 20
N_ITER = 2
P_PAD = 10240
C_PAD = 2048
CHUNK = 128

_info = plsc.get_sparse_core_info()
_GDEPTH = 3
_NC = _info.num_cores
_NS = _info.num_subcores
_NW = _NC * _NS


def _make_sc_gather(n_rows_table, n_idx, d):
    per_w = n_idx // _NW
    n_chunks = per_w // CHUNK
    mesh = plsc.VectorSubcoreMesh(core_axis_name="c", subcore_axis_name="s")

    rpt = (-(-n_rows_table // _NS) + 7) // 8 * 8
    stage = True
    nbuf = 6 if n_rows_table * d <= 4096 * FEAT else 2
    gdepth = min(_GDEPTH, nbuf - 1)

    scratch = [pltpu.VMEM((per_w,), jnp.int32)]
    scratch += [pltpu.VMEM((CHUNK, d), jnp.float32) for _ in range(nbuf)]
    if stage:
        scratch += [pltpu.VMEM_SHARED((n_rows_table, d), jnp.float32)]
    scratch += [pltpu.SemaphoreType.DMA for _ in range(2 * nbuf)]

    @functools.partial(
        pl.kernel,
        mesh=mesh,
        out_type=jax.ShapeDtypeStruct((n_idx, d), jnp.float32),
        scratch_types=scratch,
    )
    def gather_kernel(table_hbm, idx_hbm, out_hbm, idx_v, *bufs_sems):
        bufs = bufs_sems[:nbuf]
        rest = bufs_sems[nbuf:]
        if stage:
            shared, rest = rest[0], rest[1:]
        gsems = rest[:nbuf]
        ssems = rest[nbuf:]
        sid = lax.axis_index("s")
        wid = sid * _NC + lax.axis_index("c")
        base = wid * per_w
        if stage:
            row0 = jnp.minimum(sid * rpt, n_rows_table - rpt)
            pltpu.sync_copy(table_hbm.at[pl.ds(row0, rpt)],
                            shared.at[pl.ds(row0, rpt)])
        pltpu.sync_copy(idx_hbm.at[pl.ds(base, per_w)], idx_v)
        if stage:
            plsc.subcore_barrier()
            table_src = shared
        else:
            table_src = table_hbm
        gh = [None] * nbuf
        sh = [None] * nbuf
        for c in range(n_chunks):
            b = c % nbuf
            if c >= nbuf:
                sh[b].wait()
            gh[b] = pltpu.async_copy(
                table_src.at[idx_v.at[pl.ds(c * CHUNK, CHUNK)]], bufs[b], gsems[b])
            if c >= gdepth:
                pb = (c - gdepth) % nbuf
                gh[pb].wait()
                sh[pb] = pltpu.async_copy(
                    bufs[pb], out_hbm.at[pl.ds(base + (c - gdepth) * CHUNK, CHUNK)],
                    ssems[pb])
        for c in range(max(0, n_chunks - gdepth), n_chunks):
            b = c % nbuf
            gh[b].wait()
            sh[b] = pltpu.async_copy(
                bufs[b], out_hbm.at[pl.ds(base + c * CHUNK, CHUNK)], ssems[b])
        for c in range(max(0, n_chunks - nbuf), n_chunks):
            sh[c % nbuf].wait()

    return gather_kernel


_GATHER_CACHE = {}


def _sc_gather(table, idx):
    key = (table.shape[0], idx.shape[0], table.shape[1])
    if key not in _GATHER_CACHE:
        _GATHER_CACHE[key] = _make_sc_gather(*key)
    return _GATHER_CACHE[key](table, idx)


def _gru_gates(gi, gh, h):
    r = jax.nn.sigmoid(gi[:, 0:FEAT] + gh[:, 0:FEAT])
    z = jax.nn.sigmoid(gi[:, FEAT:2 * FEAT] + gh[:, FEAT:2 * FEAT])
    n = jnp.tanh(gi[:, 2 * FEAT:] + r * gh[:, 2 * FEAT:])
    return (1.0 - z) * n + z * h


def _path_body(x_ref, h_ref, wih_ref, whh_ref, bih_ref, bhh_ref, out_ref):
    h = h_ref[...]
    wih = wih_ref[...].astype(jnp.bfloat16)
    whh = whh_ref[...].astype(jnp.bfloat16)
    bih = bih_ref[...]
    bhh = bhh_ref[...]
    for s in range(DEG_P):
        gi = jnp.dot(x_ref[s].astype(jnp.bfloat16), wih,
                     preferred_element_type=jnp.float32) + bih
        gh = jnp.dot(h.astype(jnp.bfloat16), whh,
                     preferred_element_type=jnp.float32) + bhh
        h = _gru_gates(gi, gh, h)
    out_ref[...] = h


def _tc_path(x, h, wih, whh, bih, bhh, block, n_out=P_PAD):
    grid = (n_out // block,)
    return pl.pallas_call(
        _path_body,
        grid=grid,
        in_specs=[
            pl.BlockSpec((DEG_P, block, FEAT), lambda i: (0, i, 0)),
            pl.BlockSpec((block, FEAT), lambda i: (i, 0)),
            pl.BlockSpec((FEAT, GATE3), lambda i: (0, 0)),
            pl.BlockSpec((FEAT, GATE3), lambda i: (0, 0)),
            pl.BlockSpec((1, GATE3), lambda i: (0, 0)),
            pl.BlockSpec((1, GATE3), lambda i: (0, 0)),
        ],
        out_specs=pl.BlockSpec((block, FEAT), lambda i: (i, 0)),
        out_shape=jax.ShapeDtypeStruct((n_out, FEAT), jnp.float32),
    )(x, h, wih, whh, bih, bhh)


def _chan_body(pg_ref, h_ref, wih_ref, whh_ref, bih_ref, bhh_ref, out_ref):
    agg = pg_ref[0]
    for k in range(1, DEG_C):
        agg = agg + pg_ref[k]
    h = h_ref[...]
    gi = jnp.dot(agg.astype(jnp.bfloat16), wih_ref[...].astype(jnp.bfloat16),
                 preferred_element_type=jnp.float32) + bih_ref[...]
    gh = jnp.dot(h.astype(jnp.bfloat16), whh_ref[...].astype(jnp.bfloat16),
                 preferred_element_type=jnp.float32) + bhh_ref[...]
    out_ref[...] = _gru_gates(gi, gh, h)


def _tc_chan(pg, h, wih, whh, bih, bhh, block, n_out=C_PAD):
    grid = (n_out // block,)
    return pl.pallas_call(
        _chan_body,
        grid=grid,
        in_specs=[
            pl.BlockSpec((DEG_C, block, FEAT), lambda i: (0, i, 0)),
            pl.BlockSpec((block, FEAT), lambda i: (i, 0)),
            pl.BlockSpec((FEAT, GATE3), lambda i: (0, 0)),
            pl.BlockSpec((FEAT, GATE3), lambda i: (0, 0)),
            pl.BlockSpec((1, GATE3), lambda i: (0, 0)),
            pl.BlockSpec((1, GATE3), lambda i: (0, 0)),
        ],
        out_specs=pl.BlockSpec((block, FEAT), lambda i: (i, 0)),
        out_shape=jax.ShapeDtypeStruct((n_out, FEAT), jnp.float32),
    )(pg, h, wih, whh, bih, bhh)


def kernel(paths, channels, path_to_channel, channel_to_path,
           W_ih1, W_hh1, b_ih1, b_hh1, W_ih2, W_hh2, b_ih2, b_hh2):
    n_paths, feat = paths.shape
    n_chan = channels.shape[0]

    hp = jnp.pad(paths.astype(jnp.float32), ((0, P_PAD - n_paths), (0, 0)))
    hc = jnp.pad(channels.astype(jnp.float32), ((0, C_PAD - n_chan), (0, 0)))

    ptc = jnp.pad(path_to_channel.astype(jnp.int32), ((0, P_PAD - n_paths), (0, 0)))
    ctp = jnp.pad(channel_to_path.astype(jnp.int32), ((0, C_PAD - n_chan), (0, 0)))
    ptc_flat = ptc.T.reshape(-1)
    ctp_flat = ctp.T.reshape(-1)

    wih1 = W_ih1.T
    whh1 = W_hh1.T
    wih2 = W_ih2.T
    whh2 = W_hh2.T
    bih1 = b_ih1.reshape(1, -1)
    bhh1 = b_hh1.reshape(1, -1)
    bih2 = b_ih2.reshape(1, -1)
    bhh2 = b_hh2.reshape(1, -1)

    for it in range(N_ITER):
        last = it == N_ITER - 1
        xc = _sc_gather(hc, ptc_flat)
        hp = _tc_path(xc.reshape(DEG_P, P_PAD, FEAT), hp,
                      wih1, whh1, bih1, bhh1, block=1024)
        pg = _sc_gather(hp, ctp_flat)
        hc = _tc_chan(pg.reshape(DEG_C, C_PAD, FEAT), hc,
                      wih2, whh2, bih2, bhh2,
                      block=400 if last else 1024,
                      n_out=n_chan if last else C_PAD)

    return hp[:n_paths], hc


# speedup vs baseline: 1.0028x; 1.0028x over previous
import functools

import jax
import jax.numpy as jnp
from jax import lax
from jax.experimental import pallas as pl
from jax.experimental.pallas import tpu as pltpu
from jax.experimental.pallas import tpu_sc as plsc

FEAT = 128
GATE3 = 3 * FEAT
DEG_P = 4
DEG_C = 20
N_ITER = 2
P_PAD = 10240
C_PAD = 2048
CHUNK = 128

_info = plsc.get_sparse_core_info()
_GDEPTH = 3
_NC = _info.num_cores
_NS = _info.num_subcores
_NW = _NC * _NS


def _make_sc_gather(n_rows_table, n_idx, d):
    per_w = n_idx // _NW
    n_chunks = per_w // CHUNK
    mesh = plsc.VectorSubcoreMesh(core_axis_name="c", subcore_axis_name="s")

    rpt = (-(-n_rows_table // _NS) + 7) // 8 * 8
    stage = True
    nbuf = 6 if n_rows_table * d <= 4096 * FEAT else 2
    gdepth = min(_GDEPTH, nbuf - 1)

    scratch = [pltpu.VMEM((per_w,), jnp.int32)]
    scratch += [pltpu.VMEM((CHUNK, d), jnp.float32) for _ in range(nbuf)]
    if stage:
        scratch += [pltpu.VMEM_SHARED((n_rows_table, d), jnp.float32)]
    scratch += [pltpu.SemaphoreType.DMA for _ in range(2 * nbuf)]

    @functools.partial(
        pl.kernel,
        mesh=mesh,
        out_type=jax.ShapeDtypeStruct((n_idx, d), jnp.float32),
        scratch_types=scratch,
    )
    def gather_kernel(table_hbm, idx_hbm, out_hbm, idx_v, *bufs_sems):
        bufs = bufs_sems[:nbuf]
        rest = bufs_sems[nbuf:]
        if stage:
            shared, rest = rest[0], rest[1:]
        gsems = rest[:nbuf]
        ssems = rest[nbuf:]
        sid = lax.axis_index("s")
        wid = sid * _NC + lax.axis_index("c")
        base = wid * per_w
        if stage:
            row0 = jnp.minimum(sid * rpt, n_rows_table - rpt)
            pltpu.sync_copy(table_hbm.at[pl.ds(row0, rpt)],
                            shared.at[pl.ds(row0, rpt)])
        pltpu.sync_copy(idx_hbm.at[pl.ds(base, per_w)], idx_v)
        if stage:
            plsc.subcore_barrier()
            table_src = shared
        else:
            table_src = table_hbm
        gh = [None] * nbuf
        sh = [None] * nbuf
        for c in range(n_chunks):
            b = c % nbuf
            if c >= nbuf:
                sh[b].wait()
            gh[b] = pltpu.async_copy(
                table_src.at[idx_v.at[pl.ds(c * CHUNK, CHUNK)]], bufs[b], gsems[b])
            if c >= gdepth:
                pb = (c - gdepth) % nbuf
                gh[pb].wait()
                sh[pb] = pltpu.async_copy(
                    bufs[pb], out_hbm.at[pl.ds(base + (c - gdepth) * CHUNK, CHUNK)],
                    ssems[pb])
        for c in range(max(0, n_chunks - gdepth), n_chunks):
            b = c % nbuf
            gh[b].wait()
            sh[b] = pltpu.async_copy(
                bufs[b], out_hbm.at[pl.ds(base + c * CHUNK, CHUNK)], ssems[b])
        for c in range(max(0, n_chunks - nbuf), n_chunks):
            sh[c % nbuf].wait()

    return gather_kernel


_GATHER_CACHE = {}


def _sc_gather(table, idx):
    key = (table.shape[0], idx.shape[0], table.shape[1])
    if key not in _GATHER_CACHE:
        _GATHER_CACHE[key] = _make_sc_gather(*key)
    return _GATHER_CACHE[key](table, idx)


def _gru_gates(gi, gh, h):
    r = jax.nn.sigmoid(gi[:, 0:FEAT] + gh[:, 0:FEAT])
    z = jax.nn.sigmoid(gi[:, FEAT:2 * FEAT] + gh[:, FEAT:2 * FEAT])
    n = jnp.tanh(gi[:, 2 * FEAT:] + r * gh[:, 2 * FEAT:])
    return (1.0 - z) * n + z * h


def _path_body(x_ref, h_ref, wih_ref, whh_ref, bih_ref, bhh_ref, out_ref):
    h = h_ref[...]
    wih = wih_ref[...].astype(jnp.bfloat16)
    whh = whh_ref[...].astype(jnp.bfloat16)
    bih = bih_ref[...]
    bhh = bhh_ref[...]
    for s in range(DEG_P):
        gi = jnp.dot(x_ref[s].astype(jnp.bfloat16), wih,
                     preferred_element_type=jnp.float32) + bih
        gh = jnp.dot(h.astype(jnp.bfloat16), whh,
                     preferred_element_type=jnp.float32) + bhh
        h = _gru_gates(gi, gh, h)
    out_ref[...] = h


def _tc_path(x, h, wih, whh, bih, bhh, block, n_out=P_PAD):
    grid = (n_out // block,)
    return pl.pallas_call(
        _path_body,
        grid=grid,
        in_specs=[
            pl.BlockSpec((DEG_P, block, FEAT), lambda i: (0, i, 0)),
            pl.BlockSpec((block, FEAT), lambda i: (i, 0)),
            pl.BlockSpec((FEAT, GATE3), lambda i: (0, 0)),
            pl.BlockSpec((FEAT, GATE3), lambda i: (0, 0)),
            pl.BlockSpec((1, GATE3), lambda i: (0, 0)),
            pl.BlockSpec((1, GATE3), lambda i: (0, 0)),
        ],
        out_specs=pl.BlockSpec((block, FEAT), lambda i: (i, 0)),
        out_shape=jax.ShapeDtypeStruct((n_out, FEAT), jnp.float32),
    )(x, h, wih, whh, bih, bhh)


def _chan_body(pg_ref, h_ref, wih_ref, whh_ref, bih_ref, bhh_ref, out_ref):
    agg = pg_ref[0]
    for k in range(1, DEG_C):
        agg = agg + pg_ref[k]
    h = h_ref[...]
    gi = jnp.dot(agg.astype(jnp.bfloat16), wih_ref[...].astype(jnp.bfloat16),
                 preferred_element_type=jnp.float32) + bih_ref[...]
    gh = jnp.dot(h.astype(jnp.bfloat16), whh_ref[...].astype(jnp.bfloat16),
                 preferred_element_type=jnp.float32) + bhh_ref[...]
    out_ref[...] = _gru_gates(gi, gh, h)


def _tc_chan(pg, h, wih, whh, bih, bhh, block, n_out=C_PAD):
    grid = (n_out // block,)
    return pl.pallas_call(
        _chan_body,
        grid=grid,
        in_specs=[
            pl.BlockSpec((DEG_C, block, FEAT), lambda i: (0, i, 0)),
            pl.BlockSpec((block, FEAT), lambda i: (i, 0)),
            pl.BlockSpec((FEAT, GATE3), lambda i: (0, 0)),
            pl.BlockSpec((FEAT, GATE3), lambda i: (0, 0)),
            pl.BlockSpec((1, GATE3), lambda i: (0, 0)),
            pl.BlockSpec((1, GATE3), lambda i: (0, 0)),
        ],
        out_specs=pl.BlockSpec((block, FEAT), lambda i: (i, 0)),
        out_shape=jax.ShapeDtypeStruct((n_out, FEAT), jnp.float32),
    )(pg, h, wih, whh, bih, bhh)


def kernel(paths, channels, path_to_channel, channel_to_path,
           W_ih1, W_hh1, b_ih1, b_hh1, W_ih2, W_hh2, b_ih2, b_hh2):
    n_paths, feat = paths.shape
    n_chan = channels.shape[0]

    hp = jnp.pad(paths.astype(jnp.float32), ((0, P_PAD - n_paths), (0, 0)))
    hc = jnp.pad(channels.astype(jnp.float32), ((0, C_PAD - n_chan), (0, 0)))

    ptc = jnp.pad(path_to_channel.astype(jnp.int32), ((0, P_PAD - n_paths), (0, 0)))
    ctp = jnp.pad(channel_to_path.astype(jnp.int32), ((0, C_PAD - n_chan), (0, 0)))
    ptc_flat = ptc.T.reshape(-1)
    ctp_flat = ctp.T.reshape(-1)

    wih1 = W_ih1.T
    whh1 = W_hh1.T
    wih2 = W_ih2.T
    whh2 = W_hh2.T
    bih1 = b_ih1.reshape(1, -1)
    bhh1 = b_hh1.reshape(1, -1)
    bih2 = b_ih2.reshape(1, -1)
    bhh2 = b_hh2.reshape(1, -1)

    for it in range(N_ITER):
        last = it == N_ITER - 1
        xc = _sc_gather(hc, ptc_flat)
        hp = _tc_path(xc.reshape(DEG_P, P_PAD, FEAT), hp,
                      wih1, whh1, bih1, bhh1, block=1024)
        pg = _sc_gather(hp, ctp_flat)
        hc = _tc_chan(pg.reshape(DEG_C, C_PAD, FEAT), hc,
                      wih2, whh2, bih2, bhh2,
                      block=400 if last else 512,
                      n_out=n_chan if last else C_PAD)

    return hp[:n_paths], hc
